# Initial kernel scaffold; baseline (speedup 1.0000x reference)
import jax
import jax.numpy as jnp
from jax.experimental import pallas as pl


def kernel(x, edge_index, edge_attr, batch, params):
    raise NotImplementedError("write your pallas kernel here")


# speedup vs baseline: 2.5093x; 2.5093x over previous
import functools

import jax
import jax.numpy as jnp
from jax import lax
from jax.experimental import pallas as pl
from jax.experimental.pallas import tpu as pltpu
from jax.experimental.pallas import tpu_sc as plsc

N, E, D, DE, H, G = 10000, 320000, 128, 4, 128, 64
BN = 2000
BE = 8000
BN_SCALE = float(1.0 / (1.0 + 1e-5) ** 0.5)

NC, NS = 2, 16
NW = NC * NS
EPW = E // NW
CH = 80
NCH = EPW // CH
RPT = 624


def _mm(a, b):
    return lax.dot_general(a, b, (((1,), (0,)), ((), ())),
                           preferred_element_type=jnp.float32)


def _p_body(h_ref, wsn_ref, bsn_ref, wm1_ref, bm_ref, p_ref):
    t = _mm(h_ref[...], wsn_ref[...]) + bsn_ref[...]
    p_ref[...] = _mm(t, wm1_ref[...]) + bm_ref[...]


def _p_kernel(h, wsn, bsn, wm1, bm):
    return pl.pallas_call(
        _p_body,
        grid=(N // BN,),
        in_specs=[
            pl.BlockSpec((BN, D), lambda i: (i, 0)),
            pl.BlockSpec((D, H), lambda i: (0, 0)),
            pl.BlockSpec((1, H), lambda i: (0, 0)),
            pl.BlockSpec((H, H), lambda i: (0, 0)),
            pl.BlockSpec((1, H), lambda i: (0, 0)),
        ],
        out_specs=pl.BlockSpec((BN, H), lambda i: (i, 0)),
        out_shape=jax.ShapeDtypeStruct((N, H), jnp.float32),
    )(h, wsn, bsn, wm1, bm)


def _q_body(ea_ref, *refs):
    ea = ea_ref[...]
    for l in range(3):
        we, be, wm2, q = refs[3 * l], refs[3 * l + 1], refs[3 * l + 2], refs[9 + l]
        ep = _mm(ea, we[...]) + be[...]
        q[...] = _mm(ep, wm2[...])


def _q_kernel(ea, ws):
    wspecs = []
    for _ in range(3):
        wspecs += [pl.BlockSpec((DE, H), lambda i: (0, 0)),
                   pl.BlockSpec((1, H), lambda i: (0, 0)),
                   pl.BlockSpec((H, H), lambda i: (0, 0))]
    return pl.pallas_call(
        _q_body,
        grid=(E // BE,),
        in_specs=[pl.BlockSpec((BE, DE), lambda i: (i, 0))] + wspecs,
        out_specs=[pl.BlockSpec((BE, H), lambda i: (i, 0))] * 3,
        out_shape=[jax.ShapeDtypeStruct((E, H), jnp.float32)] * 3,
    )(ea, *ws)


def _sc_aggr_body(p_hbm, q_hbm, src_hbm, dst_hbm, zero_hbm, out_hbm,
                  srcv, dstv, prow, qrow, acc, gsem):
    c = lax.axis_index("c")
    s = lax.axis_index("s")
    wid = s * NC + c

    pltpu.sync_copy(zero_hbm.at[pl.ds(s * RPT, RPT)], acc.at[pl.ds(s * RPT, RPT)])

    @pl.when(s == NS - 1)
    def _():
        rem = N - NS * RPT
        pltpu.sync_copy(zero_hbm.at[pl.ds(NS * RPT, rem)],
                        acc.at[pl.ds(NS * RPT, rem)])

    plsc.subcore_barrier()

    base = wid * EPW

    def chunk(i, carry):
        eb = base + i * CH
        pltpu.sync_copy(src_hbm.at[pl.ds(eb, CH)], srcv)
        pltpu.sync_copy(dst_hbm.at[pl.ds(eb, CH)], dstv)
        pltpu.async_copy(p_hbm.at[srcv], prow, gsem).wait()
        pltpu.sync_copy(q_hbm.at[pl.ds(eb, CH)], qrow)

        def row(r, carry2):
            for j in range(H // 16):
                sl = pl.ds(j * 16, 16)
                qrow[r, sl] = jnp.maximum(prow[r, sl] + qrow[r, sl], 0.0)
            return carry2

        lax.fori_loop(0, CH, row, 0)
        pltpu.sync_copy(qrow, acc.at[dstv], add=True)
        return carry

    lax.fori_loop(0, NCH, chunk, 0)
    plsc.subcore_barrier()

    pltpu.sync_copy(acc.at[pl.ds(s * RPT, RPT)],
                    out_hbm.at[pl.ds(c * N + s * RPT, RPT)])

    @pl.when(s == NS - 1)
    def _():
        rem = N - NS * RPT
        pltpu.sync_copy(acc.at[pl.ds(NS * RPT, rem)],
                        out_hbm.at[pl.ds(c * N + NS * RPT, rem)])


def _sc_aggr(p, q, src, dst, zero):
    mesh = plsc.VectorSubcoreMesh(core_axis_name="c", subcore_axis_name="s")
    f = functools.partial(
        pl.kernel,
        out_type=jax.ShapeDtypeStruct((NC * N, H), jnp.float32),
        mesh=mesh,
        scratch_types=[
            pltpu.VMEM((CH,), jnp.int32),
            pltpu.VMEM((CH,), jnp.int32),
            pltpu.VMEM((CH, H), jnp.float32),
            pltpu.VMEM((CH, H), jnp.float32),
            pltpu.VMEM_SHARED((N, H), jnp.float32),
            pltpu.SemaphoreType.DMA,
        ],
    )(_sc_aggr_body)
    return f(p, q, src, dst, zero)


def _upd_body(use_res, h_ref, a0_ref, a1_ref, wux_ref, bux_ref,
              wg_ref, bg_ref, wu_ref, bu_ref, o_ref):
    h = h_ref[...]
    aggr = a0_ref[...] + a1_ref[...]
    old = _mm(h, wux_ref[...]) + bux_ref[...]
    wg = wg_ref[...]
    wu = wu_ref[...]
    g = _mm(old, wg[:H]) + _mm(aggr, wg[H:]) + bg_ref[...]
    gate = 1.0 / (1.0 + jnp.exp(-g))
    u = _mm(old, wu[:H]) + _mm(aggr, wu[H:]) + bu_ref[...]
    upd = jnp.maximum(u, 0.0)
    hn = gate * upd + (1.0 - gate) * old
    o = jnp.maximum(hn * BN_SCALE, 0.0)
    if use_res:
        o = o + h
    o_ref[...] = o


def _upd_kernel(h, a0, a1, p, use_res):
    return pl.pallas_call(
        functools.partial(_upd_body, use_res),
        grid=(N // BN,),
        in_specs=[
            pl.BlockSpec((BN, H), lambda i: (i, 0)),
            pl.BlockSpec((BN, H), lambda i: (i, 0)),
            pl.BlockSpec((BN, H), lambda i: (i, 0)),
            pl.BlockSpec((H, H), lambda i: (0, 0)),
            pl.BlockSpec((1, H), lambda i: (0, 0)),
            pl.BlockSpec((2 * H, H), lambda i: (0, 0)),
            pl.BlockSpec((1, H), lambda i: (0, 0)),
            pl.BlockSpec((2 * H, H), lambda i: (0, 0)),
            pl.BlockSpec((1, H), lambda i: (0, 0)),
        ],
        out_specs=pl.BlockSpec((BN, H), lambda i: (i, 0)),
        out_shape=jax.ShapeDtypeStruct((N, H), jnp.float32),
    )(h, a0, a1, p["ux"]["W"], p["ux"]["b"].reshape(1, H),
      p["g"]["W"], p["g"]["b"].reshape(1, H),
      p["u"]["W"], p["u"]["b"].reshape(1, H))


def _pool_body(x_ref, b_ref, sum_ref, max_ref, cnt_ref):
    i = pl.program_id(0)

    @pl.when(i == 0)
    def _():
        sum_ref[...] = jnp.zeros_like(sum_ref)
        max_ref[...] = jnp.full_like(max_ref, -jnp.inf)
        cnt_ref[...] = jnp.zeros_like(cnt_ref)

    x = x_ref[...]
    b = b_ref[...]
    gids = lax.broadcasted_iota(jnp.int32, (1, G), 1).astype(jnp.float32)
    onehot = (b == gids).astype(jnp.float32)
    sum_ref[...] += lax.dot_general(onehot, x, (((0,), (0,)), ((), ())),
                                    preferred_element_type=jnp.float32,
                                    precision=lax.Precision.HIGHEST)
    cnt_ref[...] += jnp.sum(onehot, axis=0, keepdims=True)

    def upd_max(g, carry):
        mask = b == jnp.float32(0) + g.astype(jnp.float32)
        xm = jnp.where(mask, x, -jnp.inf)
        m = jnp.max(xm, axis=0, keepdims=True)
        cur = max_ref[pl.ds(g, 1), :]
        max_ref[pl.ds(g, 1), :] = jnp.maximum(cur, m)
        return carry

    lax.fori_loop(0, G, upd_max, 0)


def _pool_kernel(x3, batchf):
    return pl.pallas_call(
        _pool_body,
        grid=(N // BN,),
        in_specs=[
            pl.BlockSpec((BN, H), lambda i: (i, 0)),
            pl.BlockSpec((BN, 1), lambda i: (i, 0)),
        ],
        out_specs=[
            pl.BlockSpec((G, H), lambda i: (0, 0)),
            pl.BlockSpec((G, H), lambda i: (0, 0)),
            pl.BlockSpec((1, G), lambda i: (0, 0)),
        ],
        out_shape=[
            jax.ShapeDtypeStruct((G, H), jnp.float32),
            jax.ShapeDtypeStruct((G, H), jnp.float32),
            jax.ShapeDtypeStruct((1, G), jnp.float32),
        ],
    )(x3, batchf)


def _head_body(sum_ref, max_ref, cnt_ref, w1_ref, b1_ref, w2_ref, b2_ref,
               wr_ref, br_ref, o_ref):
    s = sum_ref[...]
    cnt = cnt_ref[...].reshape(G, 1)
    mean = s / jnp.maximum(cnt, 1.0)
    comb = jnp.concatenate([mean, max_ref[...], s], axis=1)
    h1 = jnp.maximum(_mm(comb, w1_ref[...]) + b1_ref[...], 0.0)
    h2 = jnp.maximum(_mm(h1, w2_ref[...]) + b2_ref[...], 0.0)
    o_ref[...] = _mm(h2, wr_ref[...]) + br_ref[...]


def _head_kernel(sums, maxes, cnts, params):
    return pl.pallas_call(
        _head_body,
        grid=(1,),
        in_specs=[
            pl.BlockSpec((G, H), lambda i: (0, 0)),
            pl.BlockSpec((G, H), lambda i: (0, 0)),
            pl.BlockSpec((1, G), lambda i: (0, 0)),
            pl.BlockSpec((3 * H, H), lambda i: (0, 0)),
            pl.BlockSpec((1, H), lambda i: (0, 0)),
            pl.BlockSpec((H, H // 2), lambda i: (0, 0)),
            pl.BlockSpec((1, H // 2), lambda i: (0, 0)),
            pl.BlockSpec((H // 2, 1), lambda i: (0, 0)),
            pl.BlockSpec((1, 1), lambda i: (0, 0)),
        ],
        out_specs=pl.BlockSpec((G, 1), lambda i: (0, 0)),
        out_shape=jax.ShapeDtypeStruct((G, 1), jnp.float32),
    )(sums, maxes, cnts,
      params["fc1"]["W"], params["fc1"]["b"].reshape(1, H),
      params["fc2"]["W"], params["fc2"]["b"].reshape(1, H // 2),
      params["reg"]["W"], params["reg"]["b"].reshape(1, 1))


def kernel(x, edge_index, edge_attr, batch, params):
    src = edge_index[0].astype(jnp.int32)
    dst = edge_index[1].astype(jnp.int32)
    zero = jnp.zeros((N, H), jnp.float32)
    batchf = batch.astype(jnp.float32).reshape(N, 1)

    qws = []
    for l in range(3):
        p = params["conv%d" % l]
        qws += [p["e"]["W"], p["e"]["b"].reshape(1, H), p["m"]["W"][H:]]
    qs = _q_kernel(edge_attr, qws)

    h = x
    for l in range(3):
        p = params["conv%d" % l]
        pn = _p_kernel(h, p["sn"]["W"], p["sn"]["b"].reshape(1, H),
                       p["m"]["W"][:H], p["m"]["b"].reshape(1, H))
        parts = _sc_aggr(pn, qs[l], src, dst, zero)
        h = _upd_kernel(h, parts[:N], parts[N:], p, use_res=(l > 0))

    sums, maxes, cnts = _pool_kernel(h, batchf)
    out = _head_kernel(sums, maxes, cnts, params)
    return out.reshape(G)


# speedup vs baseline: 4.6463x; 1.8516x over previous
import functools

import jax
import jax.numpy as jnp
from jax import lax
from jax.experimental import pallas as pl
from jax.experimental.pallas import tpu as pltpu
from jax.experimental.pallas import tpu_sc as plsc

N, E, D, DE, H, G = 10000, 320000, 128, 4, 128, 64
BN = 2000
BE = 8000
BN_SCALE = float(1.0 / (1.0 + 1e-5) ** 0.5)

NC, NS = 2, 16
NW = NC * NS
EPW = E // NW
CH = 80
NCH = EPW // CH
RPT = 624


def _mm(a, b):
    return lax.dot_general(a, b, (((1,), (0,)), ((), ())),
                           preferred_element_type=jnp.float32)


def _p_body(h_ref, wsn_ref, bsn_ref, wm1_ref, bm_ref, p_ref):
    t = _mm(h_ref[...], wsn_ref[...]) + bsn_ref[...]
    p_ref[...] = _mm(t, wm1_ref[...]) + bm_ref[...]


def _p_kernel(h, wsn, bsn, wm1, bm):
    return pl.pallas_call(
        _p_body,
        grid=(N // BN,),
        in_specs=[
            pl.BlockSpec((BN, D), lambda i: (i, 0)),
            pl.BlockSpec((D, H), lambda i: (0, 0)),
            pl.BlockSpec((1, H), lambda i: (0, 0)),
            pl.BlockSpec((H, H), lambda i: (0, 0)),
            pl.BlockSpec((1, H), lambda i: (0, 0)),
        ],
        out_specs=pl.BlockSpec((BN, H), lambda i: (i, 0)),
        out_shape=jax.ShapeDtypeStruct((N, H), jnp.float32),
    )(h, wsn, bsn, wm1, bm)


def _q_body(ea_ref, *refs):
    ea = ea_ref[...]
    for l in range(3):
        we, be, wm2, q = refs[3 * l], refs[3 * l + 1], refs[3 * l + 2], refs[9 + l]
        ep = _mm(ea, we[...]) + be[...]
        q[...] = _mm(ep, wm2[...])


def _q_kernel(ea, ws):
    wspecs = []
    for _ in range(3):
        wspecs += [pl.BlockSpec((DE, H), lambda i: (0, 0)),
                   pl.BlockSpec((1, H), lambda i: (0, 0)),
                   pl.BlockSpec((H, H), lambda i: (0, 0))]
    return pl.pallas_call(
        _q_body,
        grid=(E // BE,),
        in_specs=[pl.BlockSpec((BE, DE), lambda i: (i, 0))] + wspecs,
        out_specs=[pl.BlockSpec((BE, H), lambda i: (i, 0))] * 3,
        out_shape=[jax.ShapeDtypeStruct((E, H), jnp.float32)] * 3,
    )(ea, *ws)


def _sc_aggr_body(p_hbm, q_hbm, src_hbm, dst_hbm, zero_hbm, out_hbm,
                  sbuf, dbuf, prow, qrow, acc, *sems):
    c = lax.axis_index("c")
    s = lax.axis_index("s")
    wid = s * NC + c
    sisem = sems[0:4]
    disem = sems[4:8]
    gsem = sems[8:10]
    qsem = sems[10:12]
    ssem = sems[12:14]

    pltpu.sync_copy(zero_hbm.at[pl.ds(s * RPT, RPT)], acc.at[pl.ds(s * RPT, RPT)])

    @pl.when(s == NS - 1)
    def _():
        rem = N - NS * RPT
        pltpu.sync_copy(zero_hbm.at[pl.ds(NS * RPT, rem)],
                        acc.at[pl.ds(NS * RPT, rem)])

    plsc.subcore_barrier()

    base = wid * EPW

    def start_idx(i, k):
        eb = base + i * CH
        pltpu.async_copy(src_hbm.at[pl.ds(eb, CH)], sbuf.at[k], sisem[k])
        pltpu.async_copy(dst_hbm.at[pl.ds(eb, CH)], dbuf.at[k], disem[k])

    def wait_idx(k):
        pltpu.make_async_copy(src_hbm.at[pl.ds(0, CH)], sbuf.at[k], sisem[k]).wait()
        pltpu.make_async_copy(dst_hbm.at[pl.ds(0, CH)], dbuf.at[k], disem[k]).wait()

    def start_in(i, k, b):
        eb = base + i * CH
        pltpu.async_copy(p_hbm.at[sbuf.at[k]], prow.at[b], gsem[b])
        pltpu.async_copy(q_hbm.at[pl.ds(eb, CH)], qrow.at[b], qsem[b])

    def wait_in(b):
        pltpu.make_async_copy(p_hbm.at[pl.ds(0, CH)], prow.at[b], gsem[b]).wait()
        pltpu.make_async_copy(q_hbm.at[pl.ds(0, CH)], qrow.at[b], qsem[b]).wait()

    def wait_sc(b):
        pltpu.make_async_copy(qrow.at[b], acc.at[pl.ds(0, CH)], ssem[b]).wait()

    start_idx(0, 0)
    start_idx(1, 1)
    wait_idx(0)
    start_in(0, 0, 0)

    def quad(jj, carry):
        for k in (0, 1, 2, 3):
            i = jj * 4 + k
            b = k % 2
            nb = 1 - b
            k1 = (k + 1) % 4
            k2 = (k + 2) % 4

            @pl.when(i < NCH)
            def _():
                wait_in(b)

                @pl.when(i + 2 < NCH)
                def _():
                    start_idx(i + 2, k2)

                @pl.when(i + 1 < NCH)
                def _():
                    wait_idx(k1)

                    @pl.when(i >= 1)
                    def _():
                        wait_sc(nb)

                    start_in(i + 1, k1, nb)

                qb = qrow.at[b]
                pb = prow.at[b]

                def row(r, carry2):
                    for j in range(H // 16):
                        sl = pl.ds(j * 16, 16)
                        qb[r, sl] = jnp.maximum(pb[r, sl] + qb[r, sl], 0.0)
                    return carry2

                lax.fori_loop(0, CH, row, 0)
                pltpu.async_copy(qb, acc.at[dbuf.at[k]], ssem[b], add=True)

        return carry

    lax.fori_loop(0, (NCH + 3) // 4, quad, 0)
    wait_sc((NCH - 2) % 2)
    wait_sc((NCH - 1) % 2)
    plsc.subcore_barrier()

    pltpu.sync_copy(acc.at[pl.ds(s * RPT, RPT)],
                    out_hbm.at[pl.ds(c * N + s * RPT, RPT)])

    @pl.when(s == NS - 1)
    def _():
        rem = N - NS * RPT
        pltpu.sync_copy(acc.at[pl.ds(NS * RPT, rem)],
                        out_hbm.at[pl.ds(c * N + NS * RPT, rem)])


def _sc_aggr(p, q, src, dst, zero):
    mesh = plsc.VectorSubcoreMesh(core_axis_name="c", subcore_axis_name="s")
    f = functools.partial(
        pl.kernel,
        out_type=jax.ShapeDtypeStruct((NC * N, H), jnp.float32),
        mesh=mesh,
        scratch_types=[
            pltpu.VMEM((4, CH), jnp.int32),
            pltpu.VMEM((4, CH), jnp.int32),
            pltpu.VMEM((2, CH, H), jnp.float32),
            pltpu.VMEM((2, CH, H), jnp.float32),
            pltpu.VMEM_SHARED((N, H), jnp.float32),
        ] + [pltpu.SemaphoreType.DMA] * 14,
    )(_sc_aggr_body)
    return f(p, q, src, dst, zero)


def _upd_body(use_res, h_ref, a0_ref, a1_ref, wux_ref, bux_ref,
              wg_ref, bg_ref, wu_ref, bu_ref, o_ref):
    h = h_ref[...]
    aggr = a0_ref[...] + a1_ref[...]
    old = _mm(h, wux_ref[...]) + bux_ref[...]
    wg = wg_ref[...]
    wu = wu_ref[...]
    g = _mm(old, wg[:H]) + _mm(aggr, wg[H:]) + bg_ref[...]
    gate = 1.0 / (1.0 + jnp.exp(-g))
    u = _mm(old, wu[:H]) + _mm(aggr, wu[H:]) + bu_ref[...]
    upd = jnp.maximum(u, 0.0)
    hn = gate * upd + (1.0 - gate) * old
    o = jnp.maximum(hn * BN_SCALE, 0.0)
    if use_res:
        o = o + h
    o_ref[...] = o


def _upd_kernel(h, a0, a1, p, use_res):
    return pl.pallas_call(
        functools.partial(_upd_body, use_res),
        grid=(N // BN,),
        in_specs=[
            pl.BlockSpec((BN, H), lambda i: (i, 0)),
            pl.BlockSpec((BN, H), lambda i: (i, 0)),
            pl.BlockSpec((BN, H), lambda i: (i, 0)),
            pl.BlockSpec((H, H), lambda i: (0, 0)),
            pl.BlockSpec((1, H), lambda i: (0, 0)),
            pl.BlockSpec((2 * H, H), lambda i: (0, 0)),
            pl.BlockSpec((1, H), lambda i: (0, 0)),
            pl.BlockSpec((2 * H, H), lambda i: (0, 0)),
            pl.BlockSpec((1, H), lambda i: (0, 0)),
        ],
        out_specs=pl.BlockSpec((BN, H), lambda i: (i, 0)),
        out_shape=jax.ShapeDtypeStruct((N, H), jnp.float32),
    )(h, a0, a1, p["ux"]["W"], p["ux"]["b"].reshape(1, H),
      p["g"]["W"], p["g"]["b"].reshape(1, H),
      p["u"]["W"], p["u"]["b"].reshape(1, H))


def _pool_body(x_ref, b_ref, sum_ref, max_ref, cnt_ref):
    i = pl.program_id(0)

    @pl.when(i == 0)
    def _():
        sum_ref[...] = jnp.zeros_like(sum_ref)
        max_ref[...] = jnp.full_like(max_ref, -jnp.inf)
        cnt_ref[...] = jnp.zeros_like(cnt_ref)

    x = x_ref[...]
    b = b_ref[...]
    gids = lax.broadcasted_iota(jnp.int32, (1, G), 1).astype(jnp.float32)
    onehot = (b == gids).astype(jnp.float32)
    sum_ref[...] += lax.dot_general(onehot, x, (((0,), (0,)), ((), ())),
                                    preferred_element_type=jnp.float32,
                                    precision=lax.Precision.HIGHEST)
    cnt_ref[...] += jnp.sum(onehot, axis=0, keepdims=True)

    def upd_max(g, carry):
        mask = b == jnp.float32(0) + g.astype(jnp.float32)
        xm = jnp.where(mask, x, -jnp.inf)
        m = jnp.max(xm, axis=0, keepdims=True)
        cur = max_ref[pl.ds(g, 1), :]
        max_ref[pl.ds(g, 1), :] = jnp.maximum(cur, m)
        return carry

    lax.fori_loop(0, G, upd_max, 0)


def _pool_kernel(x3, batchf):
    return pl.pallas_call(
        _pool_body,
        grid=(N // BN,),
        in_specs=[
            pl.BlockSpec((BN, H), lambda i: (i, 0)),
            pl.BlockSpec((BN, 1), lambda i: (i, 0)),
        ],
        out_specs=[
            pl.BlockSpec((G, H), lambda i: (0, 0)),
            pl.BlockSpec((G, H), lambda i: (0, 0)),
            pl.BlockSpec((1, G), lambda i: (0, 0)),
        ],
        out_shape=[
            jax.ShapeDtypeStruct((G, H), jnp.float32),
            jax.ShapeDtypeStruct((G, H), jnp.float32),
            jax.ShapeDtypeStruct((1, G), jnp.float32),
        ],
    )(x3, batchf)


def _head_body(sum_ref, max_ref, cnt_ref, w1_ref, b1_ref, w2_ref, b2_ref,
               wr_ref, br_ref, o_ref):
    s = sum_ref[...]
    cnt = cnt_ref[...].reshape(G, 1)
    mean = s / jnp.maximum(cnt, 1.0)
    comb = jnp.concatenate([mean, max_ref[...], s], axis=1)
    h1 = jnp.maximum(_mm(comb, w1_ref[...]) + b1_ref[...], 0.0)
    h2 = jnp.maximum(_mm(h1, w2_ref[...]) + b2_ref[...], 0.0)
    o_ref[...] = _mm(h2, wr_ref[...]) + br_ref[...]


def _head_kernel(sums, maxes, cnts, params):
    return pl.pallas_call(
        _head_body,
        grid=(1,),
        in_specs=[
            pl.BlockSpec((G, H), lambda i: (0, 0)),
            pl.BlockSpec((G, H), lambda i: (0, 0)),
            pl.BlockSpec((1, G), lambda i: (0, 0)),
            pl.BlockSpec((3 * H, H), lambda i: (0, 0)),
            pl.BlockSpec((1, H), lambda i: (0, 0)),
            pl.BlockSpec((H, H // 2), lambda i: (0, 0)),
            pl.BlockSpec((1, H // 2), lambda i: (0, 0)),
            pl.BlockSpec((H // 2, 1), lambda i: (0, 0)),
            pl.BlockSpec((1, 1), lambda i: (0, 0)),
        ],
        out_specs=pl.BlockSpec((G, 1), lambda i: (0, 0)),
        out_shape=jax.ShapeDtypeStruct((G, 1), jnp.float32),
    )(sums, maxes, cnts,
      params["fc1"]["W"], params["fc1"]["b"].reshape(1, H),
      params["fc2"]["W"], params["fc2"]["b"].reshape(1, H // 2),
      params["reg"]["W"], params["reg"]["b"].reshape(1, 1))


def kernel(x, edge_index, edge_attr, batch, params):
    src = edge_index[0].astype(jnp.int32)
    dst = edge_index[1].astype(jnp.int32)
    zero = jnp.zeros((N, H), jnp.float32)
    batchf = batch.astype(jnp.float32).reshape(N, 1)

    qws = []
    for l in range(3):
        p = params["conv%d" % l]
        qws += [p["e"]["W"], p["e"]["b"].reshape(1, H), p["m"]["W"][H:]]
    qs = _q_kernel(edge_attr, qws)

    h = x
    for l in range(3):
        p = params["conv%d" % l]
        pn = _p_kernel(h, p["sn"]["W"], p["sn"]["b"].reshape(1, H),
                       p["m"]["W"][:H], p["m"]["b"].reshape(1, H))
        parts = _sc_aggr(pn, qs[l], src, dst, zero)
        h = _upd_kernel(h, parts[:N], parts[N:], p, use_res=(l > 0))

    sums, maxes, cnts = _pool_kernel(h, batchf)
    out = _head_kernel(sums, maxes, cnts, params)
    return out.reshape(G)
